# baseline (speedup 1.0000x reference)
import jax
import jax.numpy as jnp
from jax import lax
from jax.experimental import pallas as pl

B = 128
N = 100000
BR = 8
GRID = B // BR
W = 512
NCHUNK = 195
TAIL = N - NCHUNK * W

_KS0 = 0
_KS1 = 42
_KS2 = _KS0 ^ _KS1 ^ 0x1BD11BDA

_ROT_A = (13, 15, 26, 6)
_ROT_B = (17, 29, 16, 24)


def _threefry_bits(idx):
    ks = (jnp.uint32(_KS0), jnp.uint32(_KS1), jnp.uint32(_KS2))
    x0 = jnp.full_like(idx, ks[0])
    x1 = idx + ks[1]
    for g in range(5):
        rots = _ROT_A if g % 2 == 0 else _ROT_B
        for r in rots:
            x0 = x0 + x1
            x1 = (x1 << r) | (x1 >> (32 - r))
            x1 = x1 ^ x0
        x0 = x0 + ks[(g + 1) % 3]
        x1 = x1 + ks[(g + 2) % 3] + jnp.uint32(g + 1)
    return x0 ^ x1


def _gumbel(idx):
    bits = _threefry_bits(idx)
    fb = (bits >> 9) | jnp.uint32(0x3F800000)
    u = lax.bitcast_convert_type(fb, jnp.float32) - jnp.float32(1.0)
    tiny = jnp.float32(jnp.finfo(jnp.float32).tiny)
    u = jnp.maximum(tiny, u * (jnp.float32(1.0) - tiny) + tiny)
    return -jnp.log(-jnp.log(u))


def _body(x_ref, probs_ref, y_ref):
    pid = pl.program_id(0)
    row_u = lax.broadcasted_iota(jnp.uint32, (BR, W), 0)
    col_u = lax.broadcasted_iota(jnp.uint32, (BR, W), 1)
    base = (jnp.uint32(pid) * jnp.uint32(BR) + row_u) * jnp.uint32(N) + col_u
    lane_i = lax.broadcasted_iota(jnp.int32, (BR, W), 1)

    neg_inf = jnp.float32(-jnp.inf)
    m0 = jnp.full((BR, W), neg_inf, jnp.float32)
    s0 = jnp.zeros((BR, W), jnp.float32)
    vmax0 = jnp.full((BR, W), neg_inf, jnp.float32)
    vidx0 = jnp.full((BR, W), 0x7FFFFFFF, jnp.int32)

    def step(c, carry):
        m, s, vmax, vidx = carry
        xc = x_ref[:, pl.ds(c * W, W)]
        m_new = jnp.maximum(m, xc)
        s_new = s * jnp.exp(m - m_new) + jnp.exp(xc - m_new)
        g = _gumbel(base + jnp.uint32(c * W))
        val = xc + g
        upd = val > vmax
        vmax_new = jnp.where(upd, val, vmax)
        vidx_new = jnp.where(upd, c * W + lane_i, vidx)
        return m_new, s_new, vmax_new, vidx_new

    m, s, vmax, vidx = lax.fori_loop(0, NCHUNK, step,
                                     (m0, s0, vmax0, vidx0))

    xt = x_ref[:, NCHUNK * W:N]
    t_m = jnp.max(xt, axis=1, keepdims=True)
    t_s = jnp.sum(jnp.exp(xt - t_m), axis=1, keepdims=True)
    row_t = lax.broadcasted_iota(jnp.uint32, (BR, TAIL), 0)
    col_t = lax.broadcasted_iota(jnp.uint32, (BR, TAIL), 1)
    idx_t = ((jnp.uint32(pid) * jnp.uint32(BR) + row_t) * jnp.uint32(N)
             + jnp.uint32(NCHUNK * W) + col_t)
    val_t = xt + _gumbel(idx_t)
    t_vmax = jnp.max(val_t, axis=1, keepdims=True)
    lane_t = lax.broadcasted_iota(jnp.int32, (BR, TAIL), 1)
    big = jnp.int32(0x7FFFFFFF)
    t_vidx = jnp.min(jnp.where(val_t == t_vmax, NCHUNK * W + lane_t, big),
                     axis=1, keepdims=True)

    m_l = jnp.max(m, axis=1, keepdims=True)
    M = jnp.maximum(m_l, t_m)
    S = (jnp.sum(s * jnp.exp(m - M), axis=1, keepdims=True)
         + t_s * jnp.exp(t_m - M))
    inv_s = jnp.float32(1.0) / S

    gmax = jnp.maximum(jnp.max(vmax, axis=1, keepdims=True), t_vmax)
    cand = jnp.min(jnp.where(vmax == gmax, vidx, big), axis=1, keepdims=True)
    cand_t = jnp.where(t_vmax == gmax, t_vidx, big)
    y = jnp.minimum(cand, cand_t)
    y_ref[0, 0, :] = y[:, 0]

    def storep(c, _):
        xc = x_ref[:, pl.ds(c * W, W)]
        probs_ref[:, pl.ds(c * W, W)] = jnp.exp(xc - M) * inv_s
        return 0
    lax.fori_loop(0, NCHUNK, storep, 0)
    probs_ref[:, NCHUNK * W:N] = jnp.exp(xt - M) * inv_s


@jax.jit
def kernel(x):
    probs, y3 = pl.pallas_call(
        _body,
        grid=(GRID,),
        in_specs=[pl.BlockSpec((BR, N), lambda i: (i, 0))],
        out_specs=[
            pl.BlockSpec((BR, N), lambda i: (i, 0)),
            pl.BlockSpec((1, 1, BR), lambda i: (i, 0, 0)),
        ],
        out_shape=[
            jax.ShapeDtypeStruct((B, N), jnp.float32),
            jax.ShapeDtypeStruct((GRID, 1, BR), jnp.int32),
        ],
    )(x)
    return (y3.reshape(B), probs)


# speedup vs baseline: 1.3353x; 1.3353x over previous
import jax
import jax.numpy as jnp
from jax import lax
from jax.experimental import pallas as pl

B = 128
N = 100000
BR = 8
GRID = B // BR
W = 512
NCHUNK = 195
TAIL = N - NCHUNK * W

_KS0 = 0
_KS1 = 42
_KS2 = _KS0 ^ _KS1 ^ 0x1BD11BDA

_ROT_A = (13, 15, 26, 6)
_ROT_B = (17, 29, 16, 24)


def _threefry_bits(idx):
    ks = (jnp.uint32(_KS0), jnp.uint32(_KS1), jnp.uint32(_KS2))
    x0 = jnp.full_like(idx, ks[0])
    x1 = idx + ks[1]
    for g in range(5):
        rots = _ROT_A if g % 2 == 0 else _ROT_B
        for r in rots:
            x0 = x0 + x1
            x1 = (x1 << r) | (x1 >> (32 - r))
            x1 = x1 ^ x0
        x0 = x0 + ks[(g + 1) % 3]
        x1 = x1 + ks[(g + 2) % 3] + jnp.uint32(g + 1)
    return x0 ^ x1


def _gumbel(idx):
    bits = _threefry_bits(idx)
    fb = (bits >> 9) | jnp.uint32(0x3F800000)
    u = lax.bitcast_convert_type(fb, jnp.float32) - jnp.float32(1.0)
    tiny = jnp.float32(jnp.finfo(jnp.float32).tiny)
    u = jnp.maximum(tiny, u * (jnp.float32(1.0) - tiny) + tiny)
    return -jnp.log(-jnp.log(u))


def _body(x_ref, probs_ref, y_ref):
    pid = pl.program_id(0)
    row_u = lax.broadcasted_iota(jnp.uint32, (BR, W), 0)
    col_u = lax.broadcasted_iota(jnp.uint32, (BR, W), 1)
    base = (jnp.uint32(pid) * jnp.uint32(BR) + row_u) * jnp.uint32(N) + col_u
    lane_i = lax.broadcasted_iota(jnp.int32, (BR, W), 1)

    neg_inf = jnp.float32(-jnp.inf)
    m0 = jnp.full((BR, W), neg_inf, jnp.float32)
    s0 = jnp.zeros((BR, W), jnp.float32)
    vmax0 = jnp.full((BR, W), neg_inf, jnp.float32)
    vidx0 = jnp.full((BR, W), 0x7FFFFFFF, jnp.int32)

    def update(c, carry, xc, g):
        m, s, vmax, vidx = carry
        m_new = jnp.maximum(m, xc)
        s_new = s * jnp.exp(m - m_new) + jnp.exp(xc - m_new)
        val = xc + g
        upd = val > vmax
        vmax_new = jnp.where(upd, val, vmax)
        vidx_new = jnp.where(upd, c * W + lane_i, vidx)
        return m_new, s_new, vmax_new, vidx_new

    def step2(i, carry):
        c0 = i * 2
        c1 = c0 + 1
        xc0 = x_ref[:, pl.ds(c0 * W, W)]
        xc1 = x_ref[:, pl.ds(c1 * W, W)]
        g0 = _gumbel(base + jnp.uint32(c0 * W))
        g1 = _gumbel(base + jnp.uint32(c1 * W))
        carry = update(c0, carry, xc0, g0)
        carry = update(c1, carry, xc1, g1)
        return carry

    carry = lax.fori_loop(0, NCHUNK // 2, step2, (m0, s0, vmax0, vidx0))
    cL = NCHUNK - 1
    xL = x_ref[:, pl.ds(cL * W, W)]
    m, s, vmax, vidx = update(cL, carry, xL, _gumbel(base + jnp.uint32(cL * W)))

    xt = x_ref[:, NCHUNK * W:N]
    t_m = jnp.max(xt, axis=1, keepdims=True)
    t_s = jnp.sum(jnp.exp(xt - t_m), axis=1, keepdims=True)
    row_t = lax.broadcasted_iota(jnp.uint32, (BR, TAIL), 0)
    col_t = lax.broadcasted_iota(jnp.uint32, (BR, TAIL), 1)
    idx_t = ((jnp.uint32(pid) * jnp.uint32(BR) + row_t) * jnp.uint32(N)
             + jnp.uint32(NCHUNK * W) + col_t)
    val_t = xt + _gumbel(idx_t)
    t_vmax = jnp.max(val_t, axis=1, keepdims=True)
    lane_t = lax.broadcasted_iota(jnp.int32, (BR, TAIL), 1)
    big = jnp.int32(0x7FFFFFFF)
    t_vidx = jnp.min(jnp.where(val_t == t_vmax, NCHUNK * W + lane_t, big),
                     axis=1, keepdims=True)

    m_l = jnp.max(m, axis=1, keepdims=True)
    M = jnp.maximum(m_l, t_m)
    S = (jnp.sum(s * jnp.exp(m - M), axis=1, keepdims=True)
         + t_s * jnp.exp(t_m - M))
    inv_s = jnp.float32(1.0) / S

    gmax = jnp.maximum(jnp.max(vmax, axis=1, keepdims=True), t_vmax)
    cand = jnp.min(jnp.where(vmax == gmax, vidx, big), axis=1, keepdims=True)
    cand_t = jnp.where(t_vmax == gmax, t_vidx, big)
    y = jnp.minimum(cand, cand_t)
    y_ref[0, 0, :] = y[:, 0]

    def storep(c, _):
        xc = x_ref[:, pl.ds(c * W, W)]
        probs_ref[:, pl.ds(c * W, W)] = jnp.exp(xc - M) * inv_s
        return 0
    lax.fori_loop(0, NCHUNK, storep, 0)
    probs_ref[:, NCHUNK * W:N] = jnp.exp(xt - M) * inv_s


@jax.jit
def kernel(x):
    probs, y3 = pl.pallas_call(
        _body,
        grid=(GRID,),
        in_specs=[pl.BlockSpec((BR, N), lambda i: (i, 0))],
        out_specs=[
            pl.BlockSpec((BR, N), lambda i: (i, 0)),
            pl.BlockSpec((1, 1, BR), lambda i: (i, 0, 0)),
        ],
        out_shape=[
            jax.ShapeDtypeStruct((B, N), jnp.float32),
            jax.ShapeDtypeStruct((GRID, 1, BR), jnp.int32),
        ],
    )(x)
    return (y3.reshape(B), probs)
